# baseline (speedup 1.0000x reference)
import functools

import jax
import jax.numpy as jnp
from jax import lax
from jax.experimental import pallas as pl
from jax.experimental.pallas import tpu as pltpu
from jax.experimental.pallas import tpu_sc as plsc

_B, _N, _R = 4096, 1024, 4096
_LF = 1.0 / (0.01 * 1.0e6)
_AG = _LF / 2.0
_RB = 4


def _build(B, N, R):
    info = plsc.get_sparse_core_info()
    NC, NS, L = info.num_cores, info.num_subcores, info.num_lanes
    NW = NC * NS
    rows_per = B // NW
    G = rows_per // _RB
    mesh = plsc.VectorSubcoreMesh(core_axis_name="c", subcore_axis_name="s")

    @functools.partial(
        pl.kernel,
        out_type=jax.ShapeDtypeStruct((B, 2 * R), jnp.float32),
        mesh=mesh,
        compiler_params=pltpu.CompilerParams(needs_layout_passes=False),
        scratch_types=[
            pltpu.VMEM((R,), jnp.int32),
            pltpu.VMEM((N,), jnp.int32),
            pltpu.VMEM((N,), jnp.float32),
            pltpu.VMEM((rows_per,), jnp.float32),
            pltpu.VMEM((rows_per,), jnp.float32),
            pltpu.VMEM((rows_per,), jnp.float32),
            pltpu.VMEM((rows_per,), jnp.float32),
            pltpu.VMEM((rows_per, 1), jnp.float32),
            pltpu.VMEM((rows_per,), jnp.float32),
            pltpu.VMEM((rows_per,), jnp.float32),
            pltpu.VMEM((rows_per,), jnp.float32),
            pltpu.VMEM((rows_per,), jnp.float32),
            pltpu.VMEM((_RB, N), jnp.float32),
            pltpu.VMEM((_RB, N), jnp.float32),
            pltpu.VMEM((_RB, N), jnp.float32),
            pltpu.VMEM((_RB, N), jnp.float32),
            pltpu.VMEM((_RB, 2 * R), jnp.float32),
            pltpu.VMEM((_RB, 2 * R), jnp.float32),
            pltpu.SemaphoreType.DMA,
            pltpu.SemaphoreType.DMA,
            pltpu.SemaphoreType.DMA,
            pltpu.SemaphoreType.DMA,
            pltpu.SemaphoreType.DMA,
            pltpu.SemaphoreType.DMA,
        ],
    )
    def run(rh_hbm, yin_hbm, ys_hbm, ym_hbm, dg_hbm, dl_hbm, mask_hbm, inds_hbm,
            out_hbm,
            inds_v, maski_v, mask_v, scale_v, add_v, pre_v, c2_v,
            stg_v, ys_v, ym_v, dg_v, dl_v,
            rhb0, rhb1, yib0, yib1, outb0, outb1,
            s_rh0, s_rh1, s_yi0, s_yi1, s_out0, s_out1):
        rhb = (rhb0, rhb1)
        yib = (yib0, yib1)
        outb = (outb0, outb1)
        s_rh = (s_rh0, s_rh1)
        s_yi = (s_yi0, s_yi1)
        s_out = (s_out0, s_out1)

        wid = lax.axis_index("s") * NC + lax.axis_index("c")
        base = wid * rows_per

        pltpu.sync_copy(inds_hbm, inds_v)
        pltpu.sync_copy(mask_hbm, maski_v)
        lane_iota0 = lax.iota(jnp.int32, L)
        zv0 = jnp.zeros((L,), jnp.int32)
        for src_hbm, dst_v in ((ys_hbm, ys_v), (ym_hbm, ym_v),
                               (dg_hbm, dg_v), (dl_hbm, dl_v)):
            pltpu.sync_copy(src_hbm.at[pl.ds(base, rows_per), :], stg_v)

            def unpack(c, _, dst_v=dst_v):
                dst_v[pl.ds(c * L, L)] = plsc.load_gather(
                    stg_v, [lane_iota0 + c * L, zv0])
                return 0

            lax.fori_loop(0, rows_per // L, unpack, 0, unroll=False)

        def mask_cast(c, _):
            sl = pl.ds(c * L, L)
            mask_v[sl] = maski_v[sl].astype(jnp.float32)
            return 0

        lax.fori_loop(0, N // L, mask_cast, 0, unroll=False)

        lane_iota = lax.iota(jnp.int32, L)
        zv = jnp.zeros((L,), jnp.int32)

        def prep(c, _):
            sl = pl.ds(c * L, L)
            ys = ys_v[sl]
            ym = ym_v[sl]
            dg = dg_v[sl]
            dl = dl_v[sl]
            scale = 1.0 / jnp.maximum(ym * _LF, 1.0)
            scale_v[sl] = scale
            add_v[sl] = dl / jnp.maximum(ys, ym)
            pre_v[sl] = dg * _AG
            c2_v[sl] = scale / ys
            return 0

        lax.fori_loop(0, rows_per // L, prep, 0, unroll=False)

        def start_in(g, b):
            row0 = base + g * _RB
            pltpu.async_copy(rh_hbm.at[pl.ds(row0, _RB), :], rhb[b], s_rh[b])
            pltpu.async_copy(yin_hbm.at[pl.ds(row0, _RB), :], yib[b], s_yi[b])

        def wait_in(b):
            pltpu.make_async_copy(rh_hbm.at[pl.ds(0, _RB), :], rhb[b], s_rh[b]).wait()
            pltpu.make_async_copy(yin_hbm.at[pl.ds(0, _RB), :], yib[b], s_yi[b]).wait()

        def wait_out(b):
            pltpu.make_async_copy(outb[b], out_hbm.at[pl.ds(0, _RB), :], s_out[b]).wait()

        def compute_block(g, b):
            rb, yb, ob = rhb[b], yib[b], outb[b]
            scs, ads, s2ms = [], [], []
            for r in range(_RB):
                i = g * _RB + r
                iv = jnp.full((L,), 0, jnp.int32) + i
                sc = plsc.load_gather(scale_v, [iv])
                ad = plsc.load_gather(add_v, [iv])
                pr = plsc.load_gather(pre_v, [iv])
                c2 = plsc.load_gather(c2_v, [iv])

                def dot_body(j, acc, r=r):
                    sl = pl.ds(j * L, L)
                    return acc + rb[r, sl] * yb[r, sl] * mask_v[sl]

                acc = lax.fori_loop(0, N // L, dot_body,
                                    jnp.zeros((L,), jnp.float32), unroll=4)
                s2ms.append(pr + c2 * jnp.sum(acc))
                scs.append(sc)
                ads.append(ad)

            rsplat = [jnp.full((L,), r, jnp.int32) for r in range(_RB)]

            @plsc.parallel_loop(0, R // L, unroll=4)
            def g_body(j):
                sl = pl.ds(j * L, L)
                sl2 = pl.ds(R + j * L, L)
                idx = inds_v[sl]
                for r in range(_RB):
                    gv = plsc.load_gather(rb, [rsplat[r], idx])
                    ob[r, sl] = gv * scs[r] + ads[r]
                    ob[r, sl2] = s2ms[r]

        start_in(0, 0)

        def pair(k, _):
            for b in range(2):
                g = 2 * k + b

                @pl.when(g + 1 < G)
                def _():
                    start_in(g + 1, 1 - b)

                wait_in(b)

                @pl.when(g >= 2)
                def _():
                    wait_out(b)

                compute_block(g, b)
                row0 = base + g * _RB
                pltpu.async_copy(outb[b], out_hbm.at[pl.ds(row0, _RB), :], s_out[b])
            return 0

        lax.fori_loop(0, G // 2, pair, 0, unroll=False)
        wait_out(0)
        wait_out(1)

    return run


def kernel(rate_hopping, y_in, y_surf, y_mant, dy_surf_gain, dy_surf_loss,
           inds_mant, inds_r0):
    B, N = rate_hopping.shape
    R = inds_r0.shape[0]
    run = _build(B, N, R)
    return run(
        rate_hopping,
        y_in,
        y_surf,
        y_mant,
        dy_surf_gain,
        dy_surf_loss,
        inds_mant,
        inds_r0,
    )


# speedup vs baseline: 1.1305x; 1.1305x over previous
import functools

import jax
import jax.numpy as jnp
from jax import lax
from jax.experimental import pallas as pl
from jax.experimental.pallas import tpu as pltpu
from jax.experimental.pallas import tpu_sc as plsc

_B, _N, _R = 4096, 1024, 4096
_LF = 1.0 / (0.01 * 1.0e6)
_AG = _LF / 2.0
_RB = 4


def _build(B, N, R):
    info = plsc.get_sparse_core_info()
    NC, NS, L = info.num_cores, info.num_subcores, info.num_lanes
    NW = NC * NS
    rows_per = B // NW
    G = rows_per // _RB
    mesh = plsc.VectorSubcoreMesh(core_axis_name="c", subcore_axis_name="s")

    @functools.partial(
        pl.kernel,
        out_type=jax.ShapeDtypeStruct((B, 2 * R), jnp.float32),
        mesh=mesh,
        compiler_params=pltpu.CompilerParams(needs_layout_passes=False),
        scratch_types=[
            pltpu.VMEM((R,), jnp.int32),
            pltpu.VMEM((N,), jnp.int32),
            pltpu.VMEM((N,), jnp.float32),
            pltpu.VMEM((rows_per,), jnp.float32),
            pltpu.VMEM((rows_per,), jnp.float32),
            pltpu.VMEM((rows_per,), jnp.float32),
            pltpu.VMEM((rows_per,), jnp.float32),
            pltpu.VMEM((rows_per, 4), jnp.float32),
            pltpu.VMEM((_RB, N), jnp.float32),
            pltpu.VMEM((_RB, N), jnp.float32),
            pltpu.VMEM((_RB, N), jnp.float32),
            pltpu.VMEM((_RB, N), jnp.float32),
            pltpu.VMEM((_RB, 2 * R), jnp.float32),
            pltpu.VMEM((_RB, 2 * R), jnp.float32),
            pltpu.SemaphoreType.DMA,
            pltpu.SemaphoreType.DMA,
            pltpu.SemaphoreType.DMA,
            pltpu.SemaphoreType.DMA,
            pltpu.SemaphoreType.DMA,
            pltpu.SemaphoreType.DMA,
        ],
    )
    def run(rh_hbm, yin_hbm, sc4_hbm, mask_hbm, inds_hbm,
            out_hbm,
            inds_v, maski_v, mask_v, scale_v, add_v, pre_v, c2_v,
            sc4_v,
            rhb0, rhb1, yib0, yib1, outb0, outb1,
            s_rh0, s_rh1, s_yi0, s_yi1, s_out0, s_out1):
        rhb = (rhb0, rhb1)
        yib = (yib0, yib1)
        outb = (outb0, outb1)
        s_rh = (s_rh0, s_rh1)
        s_yi = (s_yi0, s_yi1)
        s_out = (s_out0, s_out1)

        wid = lax.axis_index("s") * NC + lax.axis_index("c")
        base = wid * rows_per

        pltpu.sync_copy(inds_hbm, inds_v)
        pltpu.sync_copy(mask_hbm, maski_v)
        pltpu.sync_copy(sc4_hbm.at[pl.ds(base, rows_per), :], sc4_v)

        def mask_cast(c, _):
            sl = pl.ds(c * L, L)
            mask_v[sl] = maski_v[sl].astype(jnp.float32)
            return 0

        lax.fori_loop(0, N // L, mask_cast, 0, unroll=False)

        lane_iota = lax.iota(jnp.int32, L)
        zv = jnp.zeros((L,), jnp.int32)

        def prep(c, _):
            sl = pl.ds(c * L, L)
            cidx = lane_iota + c * L
            ys = plsc.load_gather(sc4_v, [cidx, zv])
            ym = plsc.load_gather(sc4_v, [cidx, zv + 1])
            dg = plsc.load_gather(sc4_v, [cidx, zv + 2])
            dl = plsc.load_gather(sc4_v, [cidx, zv + 3])
            scale = 1.0 / jnp.maximum(ym * _LF, 1.0)
            scale_v[sl] = scale
            add_v[sl] = dl / jnp.maximum(ys, ym)
            pre_v[sl] = dg * _AG
            c2_v[sl] = scale / ys
            return 0

        lax.fori_loop(0, rows_per // L, prep, 0, unroll=False)

        def start_in(g, b):
            row0 = base + g * _RB
            pltpu.async_copy(rh_hbm.at[pl.ds(row0, _RB), :], rhb[b], s_rh[b])
            pltpu.async_copy(yin_hbm.at[pl.ds(row0, _RB), :], yib[b], s_yi[b])

        def wait_in(b):
            pltpu.make_async_copy(rh_hbm.at[pl.ds(0, _RB), :], rhb[b], s_rh[b]).wait()
            pltpu.make_async_copy(yin_hbm.at[pl.ds(0, _RB), :], yib[b], s_yi[b]).wait()

        def wait_out(b):
            pltpu.make_async_copy(outb[b], out_hbm.at[pl.ds(0, _RB), :], s_out[b]).wait()

        def compute_block(g, b):
            rb, yb, ob = rhb[b], yib[b], outb[b]
            scs, ads, s2ms = [], [], []
            for r in range(_RB):
                i = g * _RB + r
                iv = jnp.full((L,), 0, jnp.int32) + i
                sc = plsc.load_gather(scale_v, [iv])
                ad = plsc.load_gather(add_v, [iv])
                pr = plsc.load_gather(pre_v, [iv])
                c2 = plsc.load_gather(c2_v, [iv])

                def dot_body(j, acc, r=r):
                    sl = pl.ds(j * L, L)
                    return acc + rb[r, sl] * yb[r, sl] * mask_v[sl]

                acc = lax.fori_loop(0, N // L, dot_body,
                                    jnp.zeros((L,), jnp.float32), unroll=4)
                s2ms.append(pr + c2 * jnp.sum(acc))
                scs.append(sc)
                ads.append(ad)

            rsplat = [jnp.full((L,), r, jnp.int32) for r in range(_RB)]

            @plsc.parallel_loop(0, R // L, unroll=4)
            def g_body(j):
                sl = pl.ds(j * L, L)
                sl2 = pl.ds(R + j * L, L)
                idx = inds_v[sl]
                for r in range(_RB):
                    gv = plsc.load_gather(rb, [rsplat[r], idx])
                    ob[r, sl] = gv * scs[r] + ads[r]
                    ob[r, sl2] = s2ms[r]

        start_in(0, 0)

        def pair(k, _):
            for b in range(2):
                g = 2 * k + b

                @pl.when(g + 1 < G)
                def _():
                    start_in(g + 1, 1 - b)

                wait_in(b)

                @pl.when(g >= 2)
                def _():
                    wait_out(b)

                compute_block(g, b)
                row0 = base + g * _RB
                pltpu.async_copy(outb[b], out_hbm.at[pl.ds(row0, _RB), :], s_out[b])
            return 0

        lax.fori_loop(0, G // 2, pair, 0, unroll=False)
        wait_out(0)
        wait_out(1)

    return run


def kernel(rate_hopping, y_in, y_surf, y_mant, dy_surf_gain, dy_surf_loss,
           inds_mant, inds_r0):
    B, N = rate_hopping.shape
    R = inds_r0.shape[0]
    run = _build(B, N, R)
    sc4 = jnp.concatenate([y_surf, y_mant, dy_surf_gain, dy_surf_loss], axis=1)
    return run(
        rate_hopping,
        y_in,
        sc4,
        inds_mant,
        inds_r0,
    )


# speedup vs baseline: 1.1346x; 1.0036x over previous
import functools

import jax
import jax.numpy as jnp
from jax import lax
from jax.experimental import pallas as pl
from jax.experimental.pallas import tpu as pltpu
from jax.experimental.pallas import tpu_sc as plsc

_B, _N, _R = 4096, 1024, 4096
_LF = 1.0 / (0.01 * 1.0e6)
_AG = _LF / 2.0
_RB = 4


def _build(B, N, R):
    info = plsc.get_sparse_core_info()
    NC, NS, L = info.num_cores, info.num_subcores, info.num_lanes
    NW = NC * NS
    rows_per = B // NW
    G = rows_per // _RB
    mesh = plsc.VectorSubcoreMesh(core_axis_name="c", subcore_axis_name="s")

    @functools.partial(
        pl.kernel,
        out_type=jax.ShapeDtypeStruct((B, 2 * R), jnp.float32),
        mesh=mesh,
        compiler_params=pltpu.CompilerParams(needs_layout_passes=False),
        scratch_types=[
            pltpu.VMEM((R,), jnp.int32),
            pltpu.VMEM((N,), jnp.int32),
            pltpu.VMEM((N,), jnp.float32),
            pltpu.VMEM((rows_per,), jnp.float32),
            pltpu.VMEM((rows_per,), jnp.float32),
            pltpu.VMEM((rows_per,), jnp.float32),
            pltpu.VMEM((rows_per,), jnp.float32),
            pltpu.VMEM((rows_per, 4), jnp.float32),
            pltpu.VMEM((_RB, N), jnp.float32),
            pltpu.VMEM((_RB, N), jnp.float32),
            pltpu.VMEM((_RB, N), jnp.float32),
            pltpu.VMEM((_RB, N), jnp.float32),
            pltpu.VMEM((_RB, 2 * R), jnp.float32),
            pltpu.VMEM((_RB, 2 * R), jnp.float32),
            pltpu.SemaphoreType.DMA,
            pltpu.SemaphoreType.DMA,
            pltpu.SemaphoreType.DMA,
            pltpu.SemaphoreType.DMA,
            pltpu.SemaphoreType.DMA,
            pltpu.SemaphoreType.DMA,
        ],
    )
    def run(rh_hbm, yin_hbm, sc4_hbm, mask_hbm, inds_hbm,
            out_hbm,
            inds_v, maski_v, mask_v, scale_v, add_v, pre_v, c2_v,
            sc4_v,
            rhb0, rhb1, yib0, yib1, outb0, outb1,
            s_rh0, s_rh1, s_yi0, s_yi1, s_out0, s_out1):
        rhb = (rhb0, rhb1)
        yib = (yib0, yib1)
        outb = (outb0, outb1)
        s_rh = (s_rh0, s_rh1)
        s_yi = (s_yi0, s_yi1)
        s_out = (s_out0, s_out1)

        wid = lax.axis_index("s") * NC + lax.axis_index("c")
        base = wid * rows_per

        pltpu.sync_copy(inds_hbm, inds_v)
        pltpu.sync_copy(mask_hbm, maski_v)
        pltpu.sync_copy(sc4_hbm.at[pl.ds(base, rows_per), :], sc4_v)

        def mask_cast(c, _):
            sl = pl.ds(c * L, L)
            mask_v[sl] = maski_v[sl].astype(jnp.float32)
            return 0

        lax.fori_loop(0, N // L, mask_cast, 0, unroll=False)

        lane_iota = lax.iota(jnp.int32, L)
        zv = jnp.zeros((L,), jnp.int32)

        def prep(c, _):
            sl = pl.ds(c * L, L)
            cidx = lane_iota + c * L
            ys = plsc.load_gather(sc4_v, [cidx, zv])
            ym = plsc.load_gather(sc4_v, [cidx, zv + 1])
            dg = plsc.load_gather(sc4_v, [cidx, zv + 2])
            dl = plsc.load_gather(sc4_v, [cidx, zv + 3])
            scale = 1.0 / jnp.maximum(ym * _LF, 1.0)
            scale_v[sl] = scale
            add_v[sl] = dl / jnp.maximum(ys, ym)
            pre_v[sl] = dg * _AG
            c2_v[sl] = scale / ys
            return 0

        lax.fori_loop(0, rows_per // L, prep, 0, unroll=False)

        def start_in(g, b):
            row0 = base + g * _RB
            pltpu.async_copy(rh_hbm.at[pl.ds(row0, _RB), :], rhb[b], s_rh[b])
            pltpu.async_copy(yin_hbm.at[pl.ds(row0, _RB), :], yib[b], s_yi[b])

        def wait_in(b):
            pltpu.make_async_copy(rh_hbm.at[pl.ds(0, _RB), :], rhb[b], s_rh[b]).wait()
            pltpu.make_async_copy(yin_hbm.at[pl.ds(0, _RB), :], yib[b], s_yi[b]).wait()

        def wait_out(b):
            pltpu.make_async_copy(outb[b], out_hbm.at[pl.ds(0, _RB), :], s_out[b]).wait()

        def compute_block(g, b):
            rb, yb, ob = rhb[b], yib[b], outb[b]
            scs, ads, s2ms = [], [], []
            for r in range(_RB):
                i = g * _RB + r
                iv = jnp.full((L,), 0, jnp.int32) + i
                sc = plsc.load_gather(scale_v, [iv])
                ad = plsc.load_gather(add_v, [iv])
                pr = plsc.load_gather(pre_v, [iv])
                c2 = plsc.load_gather(c2_v, [iv])

                def dot_body(j, acc, r=r):
                    sl = pl.ds(j * L, L)
                    return acc + rb[r, sl] * yb[r, sl] * mask_v[sl]

                acc = lax.fori_loop(0, N // L, dot_body,
                                    jnp.zeros((L,), jnp.float32), unroll=4)
                s2ms.append(pr + c2 * jnp.sum(acc))
                scs.append(sc)
                ads.append(ad)

            rsplat = [jnp.full((L,), r, jnp.int32) for r in range(_RB)]

            @plsc.parallel_loop(0, R // L, unroll=8)
            def g_body(j):
                sl = pl.ds(j * L, L)
                sl2 = pl.ds(R + j * L, L)
                idx = inds_v[sl]
                for r in range(_RB):
                    gv = plsc.load_gather(rb, [rsplat[r], idx])
                    ob[r, sl] = gv * scs[r] + ads[r]
                    ob[r, sl2] = s2ms[r]

        start_in(0, 0)

        def pair(k, _):
            for b in range(2):
                g = 2 * k + b

                @pl.when(g + 1 < G)
                def _():
                    start_in(g + 1, 1 - b)

                wait_in(b)

                @pl.when(g >= 2)
                def _():
                    wait_out(b)

                compute_block(g, b)
                row0 = base + g * _RB
                pltpu.async_copy(outb[b], out_hbm.at[pl.ds(row0, _RB), :], s_out[b])
            return 0

        lax.fori_loop(0, G // 2, pair, 0, unroll=False)
        wait_out(0)
        wait_out(1)

    return run


def kernel(rate_hopping, y_in, y_surf, y_mant, dy_surf_gain, dy_surf_loss,
           inds_mant, inds_r0):
    B, N = rate_hopping.shape
    R = inds_r0.shape[0]
    run = _build(B, N, R)
    sc4 = jnp.concatenate([y_surf, y_mant, dy_surf_gain, dy_surf_loss], axis=1)
    return run(
        rate_hopping,
        y_in,
        sc4,
        inds_mant,
        inds_r0,
    )
